# Initial kernel scaffold; baseline (speedup 1.0000x reference)
import jax
import jax.numpy as jnp
from jax.experimental import pallas as pl


def kernel(hidden_states, prompt_lens):
    raise NotImplementedError("write your pallas kernel here")


# speedup vs baseline: 14.6906x; 14.6906x over previous
import jax
import jax.numpy as jnp
from jax.experimental import pallas as pl

TOKENS = 32768
D = 2048
B = 16
BLK = 2048


def _pool_body(lens_row_ref, lens_col_ref, x_ref, out_ref):
    i = pl.program_id(0)
    nsteps = pl.num_programs(0)
    lens = lens_row_ref[...]
    tri = (jax.lax.broadcasted_iota(jnp.int32, (B, B), 0)
           <= jax.lax.broadcasted_iota(jnp.int32, (B, B), 1)).astype(jnp.float32)
    ends = jax.lax.dot_general(lens, tri, (((1,), (0,)), ((), ())),
                               precision=jax.lax.Precision.HIGHEST,
                               preferred_element_type=jnp.float32)
    starts = ends - lens
    rows = (i * BLK
            + jax.lax.broadcasted_iota(jnp.int32, (BLK, 1), 0)).astype(jnp.float32)
    oh = ((rows >= starts) & (rows < ends)).astype(jnp.float32)
    part = jax.lax.dot_general(
        oh, x_ref[...], (((0,), (0,)), ((), ())),
        precision=jax.lax.Precision.HIGHEST,
        preferred_element_type=jnp.float32)

    @pl.when(i == 0)
    def _():
        out_ref[...] = part

    @pl.when(i > 0)
    def _():
        out_ref[...] += part

    @pl.when(i == nsteps - 1)
    def _():
        acc = out_ref[...]
        pooled = acc / lens_col_ref[...]
        ss = jnp.sum(pooled * pooled, axis=1, keepdims=True)
        norm = jnp.maximum(jnp.sqrt(ss), 1e-12)
        out_ref[...] = pooled / norm


def kernel(hidden_states, prompt_lens):
    lens_row = prompt_lens.reshape(1, B).astype(jnp.float32)
    lens_col = prompt_lens.reshape(B, 1).astype(jnp.float32)
    grid = TOKENS // BLK
    return pl.pallas_call(
        _pool_body,
        grid=(grid,),
        in_specs=[
            pl.BlockSpec((1, B), lambda i: (0, 0)),
            pl.BlockSpec((B, 1), lambda i: (0, 0)),
            pl.BlockSpec((BLK, D), lambda i: (i, 0)),
        ],
        out_specs=pl.BlockSpec((B, D), lambda i: (0, 0)),
        out_shape=jax.ShapeDtypeStruct((B, D), jnp.float32),
    )(lens_row, lens_col, hidden_states)
